# baseline (speedup 1.0000x reference)
import jax
import jax.numpy as jnp
from jax.experimental import pallas as pl
from jax.experimental.pallas import tpu as pltpu

DIM = 1024
INTER = 512
SHARED_INTER = 2048
NUM_EXPERTS = 64
T = 32
NBUF = 5
LOOKAHEAD = 4
VLEN = 2 * NUM_EXPERTS


def _body(x_ref, gate_ref, w1_hbm, w3_hbm, w2_hbm, sg_hbm, su_hbm, sd_hbm,
          out_ref, visv_ref, viss_ref, sgb, sub, sdb, w1b, w3b, w2b,
          sems, ssems, sem_vs):
    cp_sg = pltpu.make_async_copy(sg_hbm, sgb, ssems.at[0])
    cp_su = pltpu.make_async_copy(su_hbm, sub, ssems.at[1])
    cp_sd = pltpu.make_async_copy(sd_hbm, sdb, ssems.at[2])
    cp_sg.start()
    cp_su.start()
    cp_sd.start()

    def issue(j, slot):
        e = viss_ref[0, j]
        pltpu.make_async_copy(w1_hbm.at[e], w1b.at[slot],
                              sems.at[slot, 0]).start()
        pltpu.make_async_copy(w3_hbm.at[e], w3b.at[slot],
                              sems.at[slot, 1]).start()
        pltpu.make_async_copy(w2_hbm.at[e], w2b.at[slot],
                              sems.at[slot, 2]).start()

    xv = x_ref[...]
    xb = xv.astype(jnp.bfloat16)
    scores = jnp.dot(xv, gate_ref[...].T,
                     preferred_element_type=jnp.float32)
    e_ids = jax.lax.broadcasted_iota(jnp.int32, (T, NUM_EXPERTS), 1)
    m1 = jnp.max(scores, axis=1, keepdims=True)
    a1 = jnp.min(jnp.where(scores == m1, e_ids, NUM_EXPERTS), axis=1,
                 keepdims=True)
    masked = jnp.where(e_ids == a1, -jnp.inf, scores)
    m2 = jnp.max(masked, axis=1, keepdims=True)
    a2 = jnp.min(jnp.where(masked == m2, e_ids, NUM_EXPERTS), axis=1,
                 keepdims=True)
    e2 = jnp.exp(m2 - m1)
    s1 = 1.0 / (1.0 + e2)
    s2 = e2 / (1.0 + e2)
    coef = (jnp.where(e_ids == a1, s1, 0.0)
            + jnp.where(e_ids == a2, s2, 0.0))

    act_row = (jnp.max(coef, axis=0, keepdims=True) > 0.0
               ).astype(jnp.float32)
    r64 = jax.lax.broadcasted_iota(jnp.int32, (NUM_EXPERTS, NUM_EXPERTS), 0)
    c64 = jax.lax.broadcasted_iota(jnp.int32, (NUM_EXPERTS, NUM_EXPERTS), 1)
    ident = (r64 == c64).astype(jnp.float32)
    act_col = jax.lax.dot_general(
        ident, act_row, (((1,), (1,)), ((), ())),
        preferred_element_type=jnp.float32)
    j_ge_e = (r64 >= c64).astype(jnp.float32)
    pos_col = jnp.dot(j_ge_e, act_col,
                      preferred_element_type=jnp.float32)
    n_active = jnp.max(pos_col)
    rw = jax.lax.broadcasted_iota(jnp.int32, (NUM_EXPERTS, VLEN), 0)
    cw = jax.lax.broadcasted_iota(jnp.int32, (NUM_EXPERTS, VLEN), 1)
    slot_hit = (pos_col - 1.0) == cw.astype(jnp.float32)
    visit_raw = jnp.sum(rw.astype(jnp.float32) * act_col * slot_hit,
                        axis=0, keepdims=True)
    e_col = jax.lax.broadcasted_iota(
        jnp.int32, (NUM_EXPERTS, 1), 0).astype(jnp.float32)
    last_active = jnp.max(e_col * act_col)
    j_row = jax.lax.broadcasted_iota(jnp.int32, (1, VLEN), 1)
    vis = jnp.where(j_row.astype(jnp.float32) < n_active, visit_raw,
                    last_active)
    vis = jnp.where(j_row == NUM_EXPERTS, n_active, vis)
    visv_ref[...] = vis.astype(jnp.int32)
    cp = pltpu.make_async_copy(visv_ref, viss_ref, sem_vs)
    cp.start()
    cp.wait()
    issue(0, 0)
    issue(1, 1)
    n_act = viss_ref[0, NUM_EXPERTS]

    @pl.when(n_act > 2)
    def _issue2():
        issue(2, 2)

    @pl.when(n_act > 3)
    def _issue3():
        issue(3, 3)

    cp_sg.wait()
    hg = jnp.dot(xb, sgb[...].astype(jnp.bfloat16).T,
                 preferred_element_type=jnp.float32)
    cp_su.wait()
    hu = jnp.dot(xb, sub[...].astype(jnp.bfloat16).T,
                 preferred_element_type=jnp.float32)
    h = (hg * jax.lax.logistic(hg) * hu).astype(jnp.bfloat16)
    cp_sd.wait()
    out_ref[...] = jax.lax.dot_general(
        h, sdb[...].astype(jnp.bfloat16), (((1,), (1,)), ((), ())),
        preferred_element_type=jnp.float32)

    def loop(j, carry):
        slot = jax.lax.rem(j, NBUF)
        e = viss_ref[0, j]
        pltpu.make_async_copy(w1_hbm.at[e], w1b.at[slot],
                              sems.at[slot, 0]).wait()

        @pl.when(j + LOOKAHEAD < n_act)
        def _prefetch():
            issue(j + LOOKAHEAD, jax.lax.rem(j + LOOKAHEAD, NBUF))

        h1 = jnp.dot(xb, w1b[slot].astype(jnp.bfloat16).T,
                     preferred_element_type=jnp.float32)
        pltpu.make_async_copy(w3_hbm.at[e], w3b.at[slot],
                              sems.at[slot, 1]).wait()
        h3 = jnp.dot(xb, w3b[slot].astype(jnp.bfloat16).T,
                     preferred_element_type=jnp.float32)
        pltpu.make_async_copy(w2_hbm.at[e], w2b.at[slot],
                              sems.at[slot, 2]).wait()
        g = h1 * jax.lax.logistic(h1) * h3
        c = jnp.sum(jnp.where(e_ids == e, coef, 0.0), axis=1,
                    keepdims=True)
        out_ref[...] += jnp.dot((g * c).astype(jnp.bfloat16),
                                w2b[slot].astype(jnp.bfloat16),
                                preferred_element_type=jnp.float32)
        return carry

    jax.lax.fori_loop(0, n_act, loop, 0)


@jax.jit
def kernel(x, gate_w, w1, w2, w3, shared_gate_w, shared_up_w, shared_down_w):
    orig_shape = x.shape
    x_flat = x.reshape(-1, DIM)

    out = pl.pallas_call(
        _body,
        in_specs=[
            pl.BlockSpec((T, DIM), lambda: (0, 0)),
            pl.BlockSpec((NUM_EXPERTS, DIM), lambda: (0, 0)),
            pl.BlockSpec(memory_space=pl.ANY),
            pl.BlockSpec(memory_space=pl.ANY),
            pl.BlockSpec(memory_space=pl.ANY),
            pl.BlockSpec(memory_space=pl.ANY),
            pl.BlockSpec(memory_space=pl.ANY),
            pl.BlockSpec(memory_space=pl.ANY),
        ],
        out_specs=pl.BlockSpec((T, DIM), lambda: (0, 0)),
        out_shape=jax.ShapeDtypeStruct((T, DIM), jnp.float32),
        scratch_shapes=[
            pltpu.VMEM((1, VLEN), jnp.int32),
            pltpu.SMEM((1, VLEN), jnp.int32),
            pltpu.VMEM((SHARED_INTER, DIM), jnp.float32),
            pltpu.VMEM((SHARED_INTER, DIM), jnp.float32),
            pltpu.VMEM((DIM, SHARED_INTER), jnp.float32),
            pltpu.VMEM((NBUF, INTER, DIM), jnp.float32),
            pltpu.VMEM((NBUF, INTER, DIM), jnp.float32),
            pltpu.VMEM((NBUF, INTER, DIM), jnp.float32),
            pltpu.SemaphoreType.DMA((NBUF, 3)),
            pltpu.SemaphoreType.DMA((3,)),
            pltpu.SemaphoreType.DMA,
        ],
    )(x_flat, gate_w, w1, w3, w2, shared_gate_w, shared_up_w, shared_down_w)

    return out.reshape(orig_shape)
